# baseline (speedup 1.0000x reference)
import jax
import jax.numpy as jnp
from jax import lax
from jax.experimental import pallas as pl
from jax.experimental.pallas import tpu as pltpu

B = 1024
F = 64
N = 100000
TOTAL = 2 * N
T = 2048
GRID = (TOTAL + T - 1) // T
S_TILE = N // T
LAST = GRID - 1
LN2 = 0.6931471805599453
LOG2E = 1.4426950408889634
SCALE = LOG2E / 0.05
CH = 32
NCH = T // CH
NEG = -3.0e38


def _tile_loop(lt_ref, off, row0, tgt_row, smask_bnd):
    riota = lax.broadcasted_iota(jnp.int32, (CH, 1), 0)

    def body(c, carry):
        acc_s, acc_t = carry
        x = lt_ref[pl.ds(c * CH, CH), :]
        row = riota + (row0 + c * CH)
        if smask_bnd is None:
            xs = x
        else:
            lo, hi = smask_bnd
            ok = jnp.logical_and(row >= lo, row < hi)
            xs = jnp.where(ok, x, NEG)
        p = jnp.exp2(xs - off)
        tsel = jnp.where(row == tgt_row, x, 0.0)
        for r in range(0, CH, 8):
            acc_s += p[r:r + 8, :]
            acc_t += tsel[r:r + 8, :]
        return acc_s, acc_t

    zero = jnp.zeros((8, B), jnp.float32)
    return lax.fori_loop(0, NCH, body, (zero, zero))


def _tc_body(xT_ref, f_ref, tgt_ref, cem_ref, ceh_ref,
             off_ref, sm_ref, sh_ref, tm_ref, th_ref, lt_ref):
    i = pl.program_id(0)

    @pl.when(i == 0)
    def _init():
        xT0 = xT_ref[...]
        off_ref[...] = jnp.sqrt(jnp.sum(xT0 * xT0, axis=0, keepdims=True)) - 100.0
        zero = jnp.zeros((8, B), jnp.float32)
        sm_ref[...] = zero
        sh_ref[...] = zero
        tm_ref[...] = zero
        th_ref[...] = zero

    off = off_ref[...]
    tgt = tgt_ref[0:1, :]
    lt_ref[...] = jax.lax.dot_general(
        f_ref[...], xT_ref[...],
        (((1,), (0,)), ((), ())),
        preferred_element_type=jnp.float32)

    @pl.when(i < S_TILE)
    def _mean_tile():
        s8, t8 = _tile_loop(lt_ref, off, i * T, tgt, None)
        sm_ref[...] += s8
        tm_ref[...] += t8

    @pl.when(i == S_TILE)
    def _straddle_tile():
        s8, t8 = _tile_loop(lt_ref, off, i * T, tgt, (0, N))
        sm_ref[...] += s8
        tm_ref[...] += t8
        s8h, t8h = _tile_loop(lt_ref, off, i * T, tgt + N, (N, TOTAL))
        sh_ref[...] += s8h
        th_ref[...] += t8h

    @pl.when(jnp.logical_and(i > S_TILE, i < LAST))
    def _hard_tile():
        s8, t8 = _tile_loop(lt_ref, off, i * T, tgt + N, None)
        sh_ref[...] += s8
        th_ref[...] += t8

    @pl.when(i == LAST)
    def _last_tile():
        s8, t8 = _tile_loop(lt_ref, off, i * T, tgt + N, (N, TOTAL))
        sh_ref[...] += s8
        th_ref[...] += t8
        logzm = off * LN2 + jnp.log(jnp.sum(sm_ref[...], axis=0, keepdims=True))
        logzh = off * LN2 + jnp.log(jnp.sum(sh_ref[...], axis=0, keepdims=True))
        tm = jnp.sum(tm_ref[...], axis=0, keepdims=True) * LN2
        th = jnp.sum(th_ref[...], axis=0, keepdims=True) * LN2
        cem = jnp.mean(logzm - tm)
        ceh = jnp.mean(logzh - th)
        cem_ref[...] = jnp.full((8, 128), cem, jnp.float32)
        ceh_ref[...] = jnp.full((8, 128), ceh, jnp.float32)


def _run_tc(xT, features, tgtb):
    return pl.pallas_call(
        _tc_body,
        grid=(GRID,),
        in_specs=[
            pl.BlockSpec((F, B), lambda i: (0, 0)),
            pl.BlockSpec((T, F), lambda i: (i, 0)),
            pl.BlockSpec((8, B), lambda i: (0, 0)),
        ],
        out_specs=[
            pl.BlockSpec((8, 128), lambda i: (0, 0)),
            pl.BlockSpec((8, 128), lambda i: (0, 0)),
        ],
        out_shape=[
            jax.ShapeDtypeStruct((8, 128), jnp.float32),
            jax.ShapeDtypeStruct((8, 128), jnp.float32),
        ],
        scratch_shapes=[
            pltpu.VMEM((1, B), jnp.float32),
            pltpu.VMEM((8, B), jnp.float32),
            pltpu.VMEM((8, B), jnp.float32),
            pltpu.VMEM((8, B), jnp.float32),
            pltpu.VMEM((8, B), jnp.float32),
            pltpu.VMEM((T, B), jnp.float32),
        ],
        compiler_params=pltpu.CompilerParams(
            dimension_semantics=("arbitrary",)),
    )(xT, features, tgtb)


def kernel(inputs, targets, features):
    xT = (inputs * SCALE).T
    tgtb = jnp.broadcast_to(targets.astype(jnp.int32)[None, :], (8, B))
    cem, ceh = _run_tc(xT, features, tgtb)
    ce_mean = cem[0, 0]
    ce_hard = ceh[0, 0]
    return 0.5 * (ce_hard + jnp.maximum(ce_mean - 0.2, 0.0))


# speedup vs baseline: 1.0024x; 1.0024x over previous
import jax
import jax.numpy as jnp
from jax import lax
from jax.experimental import pallas as pl
from jax.experimental.pallas import tpu as pltpu

B = 1024
F = 64
N = 100000
TOTAL = 2 * N
T = 2048
GRID = (TOTAL + T - 1) // T
S_TILE = N // T
LAST = GRID - 1
LN2 = 0.6931471805599453
LOG2E = 1.4426950408889634
SCALE = LOG2E / 0.05
CH = 256
NCH = T // CH
NEG = -3.0e38


def _tile_loop(lt_ref, off, row0, tgt_row, smask_bnd):
    riota = lax.broadcasted_iota(jnp.int32, (CH, 1), 0)

    def body(c, carry):
        acc_s, acc_t = carry
        x = lt_ref[pl.ds(c * CH, CH), :]
        row = riota + (row0 + c * CH)
        if smask_bnd is None:
            xs = x
        else:
            lo, hi = smask_bnd
            ok = jnp.logical_and(row >= lo, row < hi)
            xs = jnp.where(ok, x, NEG)
        p = jnp.exp2(xs - off)
        tsel = jnp.where(row == tgt_row, x, 0.0)
        for r in range(0, CH, 8):
            acc_s += p[r:r + 8, :]
            acc_t += tsel[r:r + 8, :]
        return acc_s, acc_t

    zero = jnp.zeros((8, B), jnp.float32)
    return lax.fori_loop(0, NCH, body, (zero, zero))


def _tc_body(xT_ref, f_ref, tgt_ref, cem_ref, ceh_ref,
             off_ref, sm_ref, sh_ref, tm_ref, th_ref, lt_ref):
    i = pl.program_id(0)

    @pl.when(i == 0)
    def _init():
        xT0 = xT_ref[...]
        off_ref[...] = jnp.sqrt(jnp.sum(xT0 * xT0, axis=0, keepdims=True)) - 100.0
        zero = jnp.zeros((8, B), jnp.float32)
        sm_ref[...] = zero
        sh_ref[...] = zero
        tm_ref[...] = zero
        th_ref[...] = zero

    off = off_ref[...]
    tgt = tgt_ref[0:1, :]
    lt_ref[...] = jax.lax.dot_general(
        f_ref[...], xT_ref[...],
        (((1,), (0,)), ((), ())),
        preferred_element_type=jnp.float32)

    @pl.when(i < S_TILE)
    def _mean_tile():
        s8, t8 = _tile_loop(lt_ref, off, i * T, tgt, None)
        sm_ref[...] += s8
        tm_ref[...] += t8

    @pl.when(i == S_TILE)
    def _straddle_tile():
        s8, t8 = _tile_loop(lt_ref, off, i * T, tgt, (0, N))
        sm_ref[...] += s8
        tm_ref[...] += t8
        s8h, t8h = _tile_loop(lt_ref, off, i * T, tgt + N, (N, TOTAL))
        sh_ref[...] += s8h
        th_ref[...] += t8h

    @pl.when(jnp.logical_and(i > S_TILE, i < LAST))
    def _hard_tile():
        s8, t8 = _tile_loop(lt_ref, off, i * T, tgt + N, None)
        sh_ref[...] += s8
        th_ref[...] += t8

    @pl.when(i == LAST)
    def _last_tile():
        s8, t8 = _tile_loop(lt_ref, off, i * T, tgt + N, (N, TOTAL))
        sh_ref[...] += s8
        th_ref[...] += t8
        logzm = off * LN2 + jnp.log(jnp.sum(sm_ref[...], axis=0, keepdims=True))
        logzh = off * LN2 + jnp.log(jnp.sum(sh_ref[...], axis=0, keepdims=True))
        tm = jnp.sum(tm_ref[...], axis=0, keepdims=True) * LN2
        th = jnp.sum(th_ref[...], axis=0, keepdims=True) * LN2
        cem = jnp.mean(logzm - tm)
        ceh = jnp.mean(logzh - th)
        cem_ref[...] = jnp.full((8, 128), cem, jnp.float32)
        ceh_ref[...] = jnp.full((8, 128), ceh, jnp.float32)


def _run_tc(xT, features, tgtb):
    return pl.pallas_call(
        _tc_body,
        grid=(GRID,),
        in_specs=[
            pl.BlockSpec((F, B), lambda i: (0, 0)),
            pl.BlockSpec((T, F), lambda i: (i, 0)),
            pl.BlockSpec((8, B), lambda i: (0, 0)),
        ],
        out_specs=[
            pl.BlockSpec((8, 128), lambda i: (0, 0)),
            pl.BlockSpec((8, 128), lambda i: (0, 0)),
        ],
        out_shape=[
            jax.ShapeDtypeStruct((8, 128), jnp.float32),
            jax.ShapeDtypeStruct((8, 128), jnp.float32),
        ],
        scratch_shapes=[
            pltpu.VMEM((1, B), jnp.float32),
            pltpu.VMEM((8, B), jnp.float32),
            pltpu.VMEM((8, B), jnp.float32),
            pltpu.VMEM((8, B), jnp.float32),
            pltpu.VMEM((8, B), jnp.float32),
            pltpu.VMEM((T, B), jnp.float32),
        ],
        compiler_params=pltpu.CompilerParams(
            dimension_semantics=("arbitrary",)),
    )(xT, features, tgtb)


def kernel(inputs, targets, features):
    xT = (inputs * SCALE).T
    tgtb = jnp.broadcast_to(targets.astype(jnp.int32)[None, :], (8, B))
    cem, ceh = _run_tc(xT, features, tgtb)
    ce_mean = cem[0, 0]
    ce_hard = ceh[0, 0]
    return 0.5 * (ce_hard + jnp.maximum(ce_mean - 0.2, 0.0))
